# baseline (speedup 1.0000x reference)
import numpy as np
import jax
import jax.numpy as jnp
from jax.experimental import pallas as pl
from jax.experimental.pallas import tpu as pltpu

SEQ_LEN = 512
PRED_LEN = 96
N_VARS = 21
D_MODEL = 128
D_FF = 256
N_HEADS = 4
N_BLOCKS = 3
PATCH_LEN = 16
STRIDE = 16
TOP_P = 0.5
ALPHA = 0.5
NUM_PATCHES = (SEQ_LEN - PATCH_LEN) // STRIDE + 1
L = N_VARS * NUM_PATCHES
DH = D_MODEL // N_HEADS
KK = int(ALPHA * L)


def _np_sinusoidal_pe(n, d, theta=10000.0):
    pos = np.arange(n)[:, None].astype(np.float64)
    i = np.arange(d)[None, :]
    angle = pos / np.power(theta, (2 * (i // 2)) / d)
    pe = np.zeros((n, d), np.float32)
    pe[:, 0::2] = np.sin(angle[:, 0::2])
    pe[:, 1::2] = np.cos(angle[:, 1::2])
    return pe


_PE672 = np.tile(_np_sinusoidal_pe(NUM_PATCHES, D_MODEL), (N_VARS, 1))

_SQRT_HALF = float(1.0 / np.sqrt(2.0))


def _gelu(x):
    return x * 0.5 * (1.0 + jax.lax.erf(x * _SQRT_HALF))


def _dot_t(a, b):
    return jax.lax.dot_general(a, b, (((1,), (1,)), ((), ())),
                               preferred_element_type=jnp.float32)


def _embed_kernel(x4_ref, pw_ref, pb_ref, pe_ref, o_ref):
    x = x4_ref[0]
    mu = jnp.mean(x, axis=(1, 2), keepdims=True)
    xc = x - mu
    mu2 = jnp.mean(xc, axis=(1, 2), keepdims=True)
    var = jnp.mean((xc - mu2) ** 2, axis=(1, 2), keepdims=True)
    xn = xc / jnp.sqrt(var + 1e-5)
    r = xn.reshape(L, PATCH_LEN)
    tok = _dot_t(r, pw_ref[...])
    o_ref[0] = tok + pb_ref[...] + pe_ref[...]


def _kth_threshold_t(keys):
    n = keys.shape[1]
    lo0 = jnp.full((1, n), np.int32(-2147483648), jnp.int32)
    hi0 = jnp.full((1, n), np.int32(2147483647), jnp.int32)

    def body(_, lh):
        lo, hi = lh
        mid = (lo >> 1) + (hi >> 1) + (lo & hi & 1)
        cnt = jnp.sum((keys <= mid).astype(jnp.int32), axis=0, keepdims=True)
        ge = cnt >= (KK + 1)
        return jnp.where(ge, lo, mid + 1), jnp.where(ge, mid, hi)

    _, hi = jax.lax.fori_loop(0, 32, body, (lo0, hi0), unroll=8)
    return hi


def _block_kernel(x_ref, n1g_ref, n1b_ref, w1_ref, b1_ref, w2_ref, b2_ref,
                  gate_ref, gcnw_ref, gcnb_ref, n2g_ref, n2b_ref,
                  f1w_ref, f1b_ref, f2w_ref, f2b_ref,
                  o_ref, s_ref, ent_ref):
    x = x_ref[0]
    mu = jnp.mean(x, axis=1, keepdims=True)
    var = jnp.mean((x - mu) ** 2, axis=1, keepdims=True)
    xn = (x - mu) / jnp.sqrt(var + 1e-5) * n1g_ref[...] + n1b_ref[...]
    xp = _dot_t(xn, gcnw_ref[...]) + gcnb_ref[...]

    LW = N_HEADS * L
    q_all = _dot_t(xn, w1_ref[...]) + b1_ref[...]
    k_all = _dot_t(xn, w2_ref[...]) + b2_ref[...]
    adj_t = _gelu(jnp.concatenate(
        [jax.lax.dot_general(k_all[:, h * DH:(h + 1) * DH],
                             q_all[:, h * DH:(h + 1) * DH],
                             (((1,), (1,)), ((), ())),
                             preferred_element_type=jnp.float32)
         for h in range(N_HEADS)], axis=1))

    ji = jax.lax.broadcasted_iota(jnp.int32, (L, LW), 0)
    lg = jax.lax.broadcasted_iota(jnp.int32, (L, LW), 1) % L
    sm_cls = ((ji % NUM_PATCHES) == (lg % NUM_PATCHES)) & (ji != lg)
    tm_cls = (ji // NUM_PATCHES) == (lg // NUM_PATCHES)
    eye = jnp.where(ji == lg, 1.0, 0.0)

    bits = jax.lax.bitcast_convert_type(adj_t, jnp.int32)
    keys = bits ^ ((bits >> 31) & np.int32(0x7FFFFFFF))
    thr = _kth_threshold_t(keys)
    masked = jnp.where(keys >= thr, adj_t, 0.0)

    gl = jax.lax.dot_general(gate_ref[...], masked, (((1,), (0,)), ((), ())),
                             preferred_element_type=jnp.float32)
    gm = jnp.max(gl, axis=0, keepdims=True)
    ge = jnp.exp(gl - gm)
    p = ge / jnp.sum(ge, axis=0, keepdims=True)
    c0, c1, c2 = p[0:1], p[1:2], p[2:3]
    r0 = (c1 > c0).astype(jnp.int32) + (c2 > c0).astype(jnp.int32)
    r1 = (c0 >= c1).astype(jnp.int32) + (c2 > c1).astype(jnp.int32)
    r2 = (c0 >= c2).astype(jnp.int32) + (c1 >= c2).astype(jnp.int32)
    s0 = jnp.where(r0 == 0, c0, 0.) + jnp.where(r1 == 0, c1, 0.) + jnp.where(r2 == 0, c2, 0.)
    s1 = jnp.where(r0 == 1, c0, 0.) + jnp.where(r1 == 1, c1, 0.) + jnp.where(r2 == 1, c2, 0.)
    s2v = jnp.where(r0 == 2, c0, 0.) + jnp.where(r1 == 2, c1, 0.) + jnp.where(r2 == 2, c2, 0.)
    keep1 = s0 <= TOP_P
    keep2 = (s0 + s1) <= TOP_P
    k1f = keep1.astype(jnp.float32)
    k2f = keep2.astype(jnp.float32)
    g0 = jnp.where(r0 == 0, 1.0, jnp.where(r0 == 1, k1f, k2f))
    g1 = jnp.where(r1 == 0, 1.0, jnp.where(r1 == 1, k1f, k2f))
    g2 = jnp.where(r2 == 0, 1.0, jnp.where(r2 == 1, k1f, k2f))
    s_acc = jnp.concatenate([s0, s1 * k1f, s2v * k2f], axis=0)
    ent_acc = jnp.sum(-p * jnp.log(p + 1e-10))

    mv = jnp.where(sm_cls, g0, jnp.where(tm_cls, g1, g2)) + eye
    a2 = masked * mv
    rmx = jnp.max(a2, axis=0, keepdims=True)
    ee = jnp.exp(a2 - rmx)
    ss = jnp.sum(ee, axis=0, keepdims=True)
    smx = ee * (1.0 / ss)
    l1 = jnp.sum(smx, axis=0, keepdims=True)
    adjn_t = smx * (1.0 / jnp.maximum(l1, 1e-12))
    outs = [jax.lax.dot_general(
        adjn_t[:, h * L:(h + 1) * L], xp[:, h * DH:(h + 1) * DH],
        (((0,), (0,)), ((), ())), preferred_element_type=jnp.float32)
        for h in range(N_HEADS)]

    out = jnp.concatenate(outs, axis=1)
    x1 = x + out
    mu2 = jnp.mean(x1, axis=1, keepdims=True)
    var2 = jnp.mean((x1 - mu2) ** 2, axis=1, keepdims=True)
    h2 = (x1 - mu2) / jnp.sqrt(var2 + 1e-5) * n2g_ref[...] + n2b_ref[...]
    ff = _dot_t(_gelu(_dot_t(h2, f1w_ref[...]) + f1b_ref[...]), f2w_ref[...]) + f2b_ref[...]
    o_ref[0] = x1 + ff
    s_ref[0] = s_acc
    e0 = jax.lax.broadcasted_iota(jnp.int32, (8, 128), 0)
    e1 = jax.lax.broadcasted_iota(jnp.int32, (8, 128), 1)
    ent_ref[0] = jnp.where((e0 == 0) & (e1 == 0), ent_acc, 0.0)


def _head_kernel(xr_ref, xe_ref, hw_ref, hb_ref, o_ref):
    xe = xe_ref[0]
    mu = jnp.mean(xe, axis=0, keepdims=True)
    xc = xe - mu
    mu2 = jnp.mean(xc, axis=0, keepdims=True)
    var = jnp.mean((xc - mu2) ** 2, axis=0, keepdims=True)
    std = jnp.sqrt(var + 1e-5)
    o = jax.lax.dot_general(hw_ref[...], xr_ref[0], (((1,), (1,)), ((), ())),
                            preferred_element_type=jnp.float32)
    o_ref[0] = (o + hb_ref[...]) * std + mu


def _loss_kernel(s_ref, e_ref, o_ref):
    tot = jnp.float32(0.0)
    n = jnp.float32(L * 3)
    for i in range(N_BLOCKS):
        sw = jnp.sum(s_ref[i], axis=0)
        s = sum(sw[:, h * L:(h + 1) * L] for h in range(N_HEADS))
        m = jnp.sum(s) / n
        v = jnp.sum((s - m) ** 2) / (n - 1.0)
        imp = v / (m * m + 1e-10)
        dyn = jnp.sum(e_ref[i]) / jnp.float32(N_HEADS * 4 * 3)
        tot = tot + imp + 0.1 * dyn
    o_ref[...] = jnp.reshape(tot / N_BLOCKS, (1, 1))


def _full(shape):
    return pl.BlockSpec(shape, lambda b: tuple(0 for _ in shape))


def kernel(x_enc, params):
    Bb = x_enc.shape[0]
    f32 = jnp.float32
    xv4 = x_enc.transpose(0, 2, 1).reshape(Bb, N_VARS, NUM_PATCHES, PATCH_LEN)

    x0 = pl.pallas_call(
        _embed_kernel,
        grid=(Bb,),
        in_specs=[pl.BlockSpec((1, N_VARS, NUM_PATCHES, PATCH_LEN), lambda b: (b, 0, 0, 0)),
                  _full((D_MODEL, PATCH_LEN)), _full((1, D_MODEL)), _full((L, D_MODEL))],
        out_specs=pl.BlockSpec((1, L, D_MODEL), lambda b: (b, 0, 0)),
        out_shape=jax.ShapeDtypeStruct((Bb, L, D_MODEL), f32),
        compiler_params=pltpu.CompilerParams(dimension_semantics=("parallel",)),
    )(xv4, params['patch_w'], params['patch_b'].reshape(1, D_MODEL), jnp.asarray(_PE672))

    x = x0
    s_list = []
    ent_list = []
    for i in range(N_BLOCKS):
        p = params['b%d' % i]
        x, s_i, e_i = pl.pallas_call(
            _block_kernel,
            grid=(Bb,),
            in_specs=[pl.BlockSpec((1, L, D_MODEL), lambda b: (b, 0, 0)),
                      _full((1, D_MODEL)), _full((1, D_MODEL)),
                      _full((D_MODEL, D_MODEL)), _full((1, D_MODEL)),
                      _full((D_MODEL, D_MODEL)), _full((1, D_MODEL)),
                      _full((3, L)),
                      _full((D_MODEL, D_MODEL)), _full((1, D_MODEL)),
                      _full((1, D_MODEL)), _full((1, D_MODEL)),
                      _full((D_FF, D_MODEL)), _full((1, D_FF)),
                      _full((D_MODEL, D_FF)), _full((1, D_MODEL))],
            out_specs=[pl.BlockSpec((1, L, D_MODEL), lambda b: (b, 0, 0)),
                       pl.BlockSpec((1, 3, N_HEADS * L), lambda b: (b, 0, 0)),
                       pl.BlockSpec((1, 8, 128), lambda b: (b, 0, 0))],
            out_shape=[jax.ShapeDtypeStruct((Bb, L, D_MODEL), f32),
                       jax.ShapeDtypeStruct((Bb, 3, N_HEADS * L), f32),
                       jax.ShapeDtypeStruct((Bb, 8, 128), f32)],
            compiler_params=pltpu.CompilerParams(
                dimension_semantics=("parallel",)),
        )(x, p['n1_g'].reshape(1, -1), p['n1_b'].reshape(1, -1),
          jax.scipy.linalg.block_diag(*([p['gl_w1']] * N_HEADS)),
          jnp.tile(p['gl_b1'], N_HEADS).reshape(1, -1),
          jax.scipy.linalg.block_diag(*([p['gl_w2']] * N_HEADS)),
          jnp.tile(p['gl_b2'], N_HEADS).reshape(1, -1),
          p['gate_w'],
          p['gcn_w'], p['gcn_b'].reshape(1, -1),
          p['n2_g'].reshape(1, -1), p['n2_b'].reshape(1, -1),
          p['f1_w'], p['f1_b'].reshape(1, -1),
          p['f2_w'], p['f2_b'].reshape(1, -1))
        s_list.append(s_i)
        ent_list.append(e_i)

    xr = x.reshape(Bb, N_VARS, NUM_PATCHES * D_MODEL)
    out = pl.pallas_call(
        _head_kernel,
        grid=(Bb,),
        in_specs=[pl.BlockSpec((1, N_VARS, NUM_PATCHES * D_MODEL), lambda b: (b, 0, 0)),
                  pl.BlockSpec((1, SEQ_LEN, N_VARS), lambda b: (b, 0, 0)),
                  _full((PRED_LEN, NUM_PATCHES * D_MODEL)), _full((PRED_LEN, 1))],
        out_specs=pl.BlockSpec((1, PRED_LEN, N_VARS), lambda b: (b, 0, 0)),
        out_shape=jax.ShapeDtypeStruct((Bb, PRED_LEN, N_VARS), f32),
        compiler_params=pltpu.CompilerParams(dimension_semantics=("parallel",)),
    )(xr, x_enc, params['head_w'], params['head_b'].reshape(PRED_LEN, 1))

    s_all = jnp.stack(s_list)
    e_all = jnp.stack(ent_list)
    loss = pl.pallas_call(
        _loss_kernel,
        grid=(1,),
        in_specs=[_full((N_BLOCKS, Bb, 3, N_HEADS * L)), _full((N_BLOCKS, Bb, 8, 128))],
        out_specs=_full((1, 1)),
        out_shape=jax.ShapeDtypeStruct((1, 1), f32),
    )(s_all, e_all)[0, 0]

    return out, loss


# speedup vs baseline: 1.1756x; 1.1756x over previous
import numpy as np
import jax
import jax.numpy as jnp
from jax.experimental import pallas as pl
from jax.experimental.pallas import tpu as pltpu

SEQ_LEN = 512
PRED_LEN = 96
N_VARS = 21
D_MODEL = 128
D_FF = 256
N_HEADS = 4
N_BLOCKS = 3
PATCH_LEN = 16
STRIDE = 16
TOP_P = 0.5
ALPHA = 0.5
NUM_PATCHES = (SEQ_LEN - PATCH_LEN) // STRIDE + 1
L = N_VARS * NUM_PATCHES
DH = D_MODEL // N_HEADS
KK = int(ALPHA * L)


def _np_sinusoidal_pe(n, d, theta=10000.0):
    pos = np.arange(n)[:, None].astype(np.float64)
    i = np.arange(d)[None, :]
    angle = pos / np.power(theta, (2 * (i // 2)) / d)
    pe = np.zeros((n, d), np.float32)
    pe[:, 0::2] = np.sin(angle[:, 0::2])
    pe[:, 1::2] = np.cos(angle[:, 1::2])
    return pe


_PE672 = np.tile(_np_sinusoidal_pe(NUM_PATCHES, D_MODEL), (N_VARS, 1))

_SQRT_HALF = float(1.0 / np.sqrt(2.0))


def _gelu(x):
    return x * 0.5 * (1.0 + jax.lax.erf(x * _SQRT_HALF))


def _dot_t(a, b):
    return jax.lax.dot_general(a, b, (((1,), (1,)), ((), ())),
                               preferred_element_type=jnp.float32)


def _embed_kernel(x4_ref, pw_ref, pb_ref, pe_ref, o_ref):
    x = x4_ref[0]
    mu = jnp.mean(x, axis=(1, 2), keepdims=True)
    xc = x - mu
    mu2 = jnp.mean(xc, axis=(1, 2), keepdims=True)
    var = jnp.mean((xc - mu2) ** 2, axis=(1, 2), keepdims=True)
    xn = xc / jnp.sqrt(var + 1e-5)
    r = xn.reshape(L, PATCH_LEN)
    tok = _dot_t(r, pw_ref[...])
    o_ref[0] = tok + pb_ref[...] + pe_ref[...]


def _kth_threshold_t(keys):
    n = keys.shape[1]
    lo0 = jnp.full((1, n), np.int32(-2147483648), jnp.int32)
    hi0 = jnp.full((1, n), np.int32(2147483647), jnp.int32)

    ones_row = jnp.ones((1, L), jnp.float32)

    def body(_, lh):
        lo, hi = lh
        mid = (lo >> 1) + (hi >> 1) + (lo & hi & 1)
        pred = (keys <= mid).astype(jnp.float32)
        cnt = jax.lax.dot_general(ones_row, pred, (((1,), (0,)), ((), ())),
                                  preferred_element_type=jnp.float32)
        ge = cnt >= jnp.float32(KK + 1)
        return jnp.where(ge, lo, mid + 1), jnp.where(ge, mid, hi)

    _, hi = jax.lax.fori_loop(0, 32, body, (lo0, hi0))
    return hi


def _block_kernel(x_ref, n1g_ref, n1b_ref, w1_ref, b1_ref, w2_ref, b2_ref,
                  gate_ref, gcnw_ref, gcnb_ref, n2g_ref, n2b_ref,
                  f1w_ref, f1b_ref, f2w_ref, f2b_ref,
                  o_ref, s_ref, ent_ref):
    x = x_ref[0]
    mu = jnp.mean(x, axis=1, keepdims=True)
    var = jnp.mean((x - mu) ** 2, axis=1, keepdims=True)
    xn = (x - mu) / jnp.sqrt(var + 1e-5) * n1g_ref[...] + n1b_ref[...]
    xp = _dot_t(xn, gcnw_ref[...]) + gcnb_ref[...]

    LW = N_HEADS * L
    q_all = _dot_t(xn, w1_ref[...]) + b1_ref[...]
    k_all = _dot_t(xn, w2_ref[...]) + b2_ref[...]
    adj_t = _gelu(jnp.concatenate(
        [jax.lax.dot_general(k_all[:, h * DH:(h + 1) * DH],
                             q_all[:, h * DH:(h + 1) * DH],
                             (((1,), (1,)), ((), ())),
                             preferred_element_type=jnp.float32)
         for h in range(N_HEADS)], axis=1))

    ji = jax.lax.broadcasted_iota(jnp.int32, (L, LW), 0)
    lg = jax.lax.broadcasted_iota(jnp.int32, (L, LW), 1) % L
    sm_cls = ((ji % NUM_PATCHES) == (lg % NUM_PATCHES)) & (ji != lg)
    tm_cls = (ji // NUM_PATCHES) == (lg // NUM_PATCHES)
    eye = jnp.where(ji == lg, 1.0, 0.0)

    bits = jax.lax.bitcast_convert_type(adj_t, jnp.int32)
    keys = bits ^ ((bits >> 31) & np.int32(0x7FFFFFFF))
    thr = _kth_threshold_t(keys)
    masked = jnp.where(keys >= thr, adj_t, 0.0)

    gl = jax.lax.dot_general(gate_ref[...], masked, (((1,), (0,)), ((), ())),
                             preferred_element_type=jnp.float32)
    gm = jnp.max(gl, axis=0, keepdims=True)
    ge = jnp.exp(gl - gm)
    p = ge / jnp.sum(ge, axis=0, keepdims=True)
    c0, c1, c2 = p[0:1], p[1:2], p[2:3]
    r0 = (c1 > c0).astype(jnp.int32) + (c2 > c0).astype(jnp.int32)
    r1 = (c0 >= c1).astype(jnp.int32) + (c2 > c1).astype(jnp.int32)
    r2 = (c0 >= c2).astype(jnp.int32) + (c1 >= c2).astype(jnp.int32)
    s0 = jnp.where(r0 == 0, c0, 0.) + jnp.where(r1 == 0, c1, 0.) + jnp.where(r2 == 0, c2, 0.)
    s1 = jnp.where(r0 == 1, c0, 0.) + jnp.where(r1 == 1, c1, 0.) + jnp.where(r2 == 1, c2, 0.)
    s2v = jnp.where(r0 == 2, c0, 0.) + jnp.where(r1 == 2, c1, 0.) + jnp.where(r2 == 2, c2, 0.)
    keep1 = s0 <= TOP_P
    keep2 = (s0 + s1) <= TOP_P
    k1f = keep1.astype(jnp.float32)
    k2f = keep2.astype(jnp.float32)
    g0 = jnp.where(r0 == 0, 1.0, jnp.where(r0 == 1, k1f, k2f))
    g1 = jnp.where(r1 == 0, 1.0, jnp.where(r1 == 1, k1f, k2f))
    g2 = jnp.where(r2 == 0, 1.0, jnp.where(r2 == 1, k1f, k2f))
    s_acc = jnp.concatenate([s0, s1 * k1f, s2v * k2f], axis=0)
    ent_acc = jnp.sum(-p * jnp.log(p + 1e-10))

    mv = jnp.where(sm_cls, g0, jnp.where(tm_cls, g1, g2)) + eye
    a2 = masked * mv
    rmx = jnp.max(a2, axis=0, keepdims=True)
    ee = jnp.exp(a2 - rmx)
    ss = jnp.sum(ee, axis=0, keepdims=True)
    smx = ee * (1.0 / ss)
    l1 = jnp.sum(smx, axis=0, keepdims=True)
    adjn_t = smx * (1.0 / jnp.maximum(l1, 1e-12))
    outs = [jax.lax.dot_general(
        adjn_t[:, h * L:(h + 1) * L], xp[:, h * DH:(h + 1) * DH],
        (((0,), (0,)), ((), ())), preferred_element_type=jnp.float32)
        for h in range(N_HEADS)]

    out = jnp.concatenate(outs, axis=1)
    x1 = x + out
    mu2 = jnp.mean(x1, axis=1, keepdims=True)
    var2 = jnp.mean((x1 - mu2) ** 2, axis=1, keepdims=True)
    h2 = (x1 - mu2) / jnp.sqrt(var2 + 1e-5) * n2g_ref[...] + n2b_ref[...]
    ff = _dot_t(_gelu(_dot_t(h2, f1w_ref[...]) + f1b_ref[...]), f2w_ref[...]) + f2b_ref[...]
    o_ref[0] = x1 + ff
    s_ref[0] = s_acc
    e0 = jax.lax.broadcasted_iota(jnp.int32, (8, 128), 0)
    e1 = jax.lax.broadcasted_iota(jnp.int32, (8, 128), 1)
    ent_ref[0] = jnp.where((e0 == 0) & (e1 == 0), ent_acc, 0.0)


def _head_kernel(xr_ref, xe_ref, hw_ref, hb_ref, o_ref):
    xe = xe_ref[0]
    mu = jnp.mean(xe, axis=0, keepdims=True)
    xc = xe - mu
    mu2 = jnp.mean(xc, axis=0, keepdims=True)
    var = jnp.mean((xc - mu2) ** 2, axis=0, keepdims=True)
    std = jnp.sqrt(var + 1e-5)
    o = jax.lax.dot_general(hw_ref[...], xr_ref[0], (((1,), (1,)), ((), ())),
                            preferred_element_type=jnp.float32)
    o_ref[0] = (o + hb_ref[...]) * std + mu


def _loss_kernel(s_ref, e_ref, o_ref):
    tot = jnp.float32(0.0)
    n = jnp.float32(L * 3)
    for i in range(N_BLOCKS):
        sw = jnp.sum(s_ref[i], axis=0)
        s = sum(sw[:, h * L:(h + 1) * L] for h in range(N_HEADS))
        m = jnp.sum(s) / n
        v = jnp.sum((s - m) ** 2) / (n - 1.0)
        imp = v / (m * m + 1e-10)
        dyn = jnp.sum(e_ref[i]) / jnp.float32(N_HEADS * 4 * 3)
        tot = tot + imp + 0.1 * dyn
    o_ref[...] = jnp.reshape(tot / N_BLOCKS, (1, 1))


def _full(shape):
    return pl.BlockSpec(shape, lambda b: tuple(0 for _ in shape))


def kernel(x_enc, params):
    Bb = x_enc.shape[0]
    f32 = jnp.float32
    xv4 = x_enc.transpose(0, 2, 1).reshape(Bb, N_VARS, NUM_PATCHES, PATCH_LEN)

    x0 = pl.pallas_call(
        _embed_kernel,
        grid=(Bb,),
        in_specs=[pl.BlockSpec((1, N_VARS, NUM_PATCHES, PATCH_LEN), lambda b: (b, 0, 0, 0)),
                  _full((D_MODEL, PATCH_LEN)), _full((1, D_MODEL)), _full((L, D_MODEL))],
        out_specs=pl.BlockSpec((1, L, D_MODEL), lambda b: (b, 0, 0)),
        out_shape=jax.ShapeDtypeStruct((Bb, L, D_MODEL), f32),
        compiler_params=pltpu.CompilerParams(dimension_semantics=("parallel",)),
    )(xv4, params['patch_w'], params['patch_b'].reshape(1, D_MODEL), jnp.asarray(_PE672))

    x = x0
    s_list = []
    ent_list = []
    for i in range(N_BLOCKS):
        p = params['b%d' % i]
        x, s_i, e_i = pl.pallas_call(
            _block_kernel,
            grid=(Bb,),
            in_specs=[pl.BlockSpec((1, L, D_MODEL), lambda b: (b, 0, 0)),
                      _full((1, D_MODEL)), _full((1, D_MODEL)),
                      _full((D_MODEL, D_MODEL)), _full((1, D_MODEL)),
                      _full((D_MODEL, D_MODEL)), _full((1, D_MODEL)),
                      _full((3, L)),
                      _full((D_MODEL, D_MODEL)), _full((1, D_MODEL)),
                      _full((1, D_MODEL)), _full((1, D_MODEL)),
                      _full((D_FF, D_MODEL)), _full((1, D_FF)),
                      _full((D_MODEL, D_FF)), _full((1, D_MODEL))],
            out_specs=[pl.BlockSpec((1, L, D_MODEL), lambda b: (b, 0, 0)),
                       pl.BlockSpec((1, 3, N_HEADS * L), lambda b: (b, 0, 0)),
                       pl.BlockSpec((1, 8, 128), lambda b: (b, 0, 0))],
            out_shape=[jax.ShapeDtypeStruct((Bb, L, D_MODEL), f32),
                       jax.ShapeDtypeStruct((Bb, 3, N_HEADS * L), f32),
                       jax.ShapeDtypeStruct((Bb, 8, 128), f32)],
            compiler_params=pltpu.CompilerParams(
                dimension_semantics=("parallel",)),
        )(x, p['n1_g'].reshape(1, -1), p['n1_b'].reshape(1, -1),
          jax.scipy.linalg.block_diag(*([p['gl_w1']] * N_HEADS)),
          jnp.tile(p['gl_b1'], N_HEADS).reshape(1, -1),
          jax.scipy.linalg.block_diag(*([p['gl_w2']] * N_HEADS)),
          jnp.tile(p['gl_b2'], N_HEADS).reshape(1, -1),
          p['gate_w'],
          p['gcn_w'], p['gcn_b'].reshape(1, -1),
          p['n2_g'].reshape(1, -1), p['n2_b'].reshape(1, -1),
          p['f1_w'], p['f1_b'].reshape(1, -1),
          p['f2_w'], p['f2_b'].reshape(1, -1))
        s_list.append(s_i)
        ent_list.append(e_i)

    xr = x.reshape(Bb, N_VARS, NUM_PATCHES * D_MODEL)
    out = pl.pallas_call(
        _head_kernel,
        grid=(Bb,),
        in_specs=[pl.BlockSpec((1, N_VARS, NUM_PATCHES * D_MODEL), lambda b: (b, 0, 0)),
                  pl.BlockSpec((1, SEQ_LEN, N_VARS), lambda b: (b, 0, 0)),
                  _full((PRED_LEN, NUM_PATCHES * D_MODEL)), _full((PRED_LEN, 1))],
        out_specs=pl.BlockSpec((1, PRED_LEN, N_VARS), lambda b: (b, 0, 0)),
        out_shape=jax.ShapeDtypeStruct((Bb, PRED_LEN, N_VARS), f32),
        compiler_params=pltpu.CompilerParams(dimension_semantics=("parallel",)),
    )(xr, x_enc, params['head_w'], params['head_b'].reshape(PRED_LEN, 1))

    s_all = jnp.stack(s_list)
    e_all = jnp.stack(ent_list)
    loss = pl.pallas_call(
        _loss_kernel,
        grid=(1,),
        in_specs=[_full((N_BLOCKS, Bb, 3, N_HEADS * L)), _full((N_BLOCKS, Bb, 8, 128))],
        out_specs=_full((1, 1)),
        out_shape=jax.ShapeDtypeStruct((1, 1), f32),
    )(s_all, e_all)[0, 0]

    return out, loss
